# Initial kernel scaffold; baseline (speedup 1.0000x reference)
import jax
import jax.numpy as jnp
from jax.experimental import pallas as pl


def kernel(x, edge_index, edge_attr, batch, edge_nn1_W, edge_nn1_b, conv1_root, conv1_bias, bn1_gamma, bn1_beta, edge_nn2_W, edge_nn2_b, conv2_root, conv2_bias, bn2_gamma, bn2_beta, mlp_W1, mlp_b1, mlp_W2, mlp_b2):
    raise NotImplementedError("write your pallas kernel here")


# speedup vs baseline: 2.6780x; 2.6780x over previous
import functools

import jax
import jax.numpy as jnp
from jax import lax
from jax.experimental import pallas as pl
from jax.experimental.pallas import tpu as pltpu
from jax.experimental.pallas import tpu_sc as plsc

NC = 2
NS = 16
NW = NC * NS
CH = 128
PW = 128
SCH = 512
EPS = 1e-05


def _vsc_mesh():
  return plsc.VectorSubcoreMesh(core_axis_name="c", subcore_axis_name="s",
                                num_cores=NC, num_subcores=NS)


def _sc_gather(table, idx2d):
  n_idx_rows, ch = idx2d.shape
  R = n_idx_rows * ch
  per_w = R // NW
  n_ch = per_w // CH
  n_sub = per_w // SCH
  ch_per_sub = SCH // CH

  @functools.partial(
      pl.kernel,
      out_type=jax.ShapeDtypeStruct((R, PW), jnp.float32),
      mesh=_vsc_mesh(),
      scratch_types=[
          pltpu.VMEM((n_ch, CH), jnp.int32),
          pltpu.VMEM((SCH, PW), jnp.float32),
          pltpu.SemaphoreType.DMA,
      ])
  def k(table_hbm, idx_hbm, out_hbm, idx_v, rows_v, sem):
    c = lax.axis_index("c")
    s = lax.axis_index("s")
    wid = c * NS + s
    pltpu.sync_copy(idx_hbm.at[pl.ds(wid * n_ch, n_ch)], idx_v)
    for k_ in range(n_sub):
      cps = []
      for j in range(ch_per_sub):
        cps.append(pltpu.async_copy(
            table_hbm.at[idx_v.at[k_ * ch_per_sub + j]],
            rows_v.at[pl.ds(j * CH, CH)], sem))
      for cp in cps:
        cp.wait()
      pltpu.sync_copy(rows_v, out_hbm.at[pl.ds(wid * per_w + k_ * SCH, SCH)])

  return k(table, idx2d)


def _sc_scatter(rows, idx_byq, nseg):
  R, W = rows.shape
  NQ = idx_byq.shape[0]
  NPH = NQ // NC
  nh = nseg // NQ
  stripe = nh // NS
  per_w = R // NS
  n_ch = per_w // CH
  n_sub = per_w // SCH
  ch_per_sub = SCH // CH
  zrows = min(16, stripe)
  zreps = stripe // zrows

  @functools.partial(
      pl.kernel,
      out_type=jax.ShapeDtypeStruct((nseg, W), jnp.float32),
      mesh=_vsc_mesh(),
      scratch_types=[
          pltpu.VMEM((n_ch, CH), jnp.int32),
          pltpu.VMEM((SCH, W), jnp.float32),
          pltpu.VMEM((16, W), jnp.float32),
          pltpu.VMEM_SHARED((nh + 8, W), jnp.float32),
          pltpu.SemaphoreType.DMA,
      ])
  def k(rows_hbm, idx_hbm, out_hbm, idx_v, rows_v, zer_v, acc, sem):
    c = lax.axis_index("c")
    s = lax.axis_index("s")

    def fill_body(i, _):
      for j in range(W // 16):
        zer_v[i, pl.ds(16 * j, 16)] = jnp.zeros((16,), jnp.float32)
      return 0
    lax.fori_loop(0, 16, fill_body, 0)

    for p in range(NPH):
      q = c * NPH + p
      for z in range(zreps):
        pltpu.sync_copy(zer_v.at[pl.ds(0, zrows)],
                        acc.at[pl.ds(s * stripe + z * zrows, zrows)])
      plsc.subcore_barrier()

      pltpu.sync_copy(idx_hbm.at[q, pl.ds(s * n_ch, n_ch)], idx_v)
      for k_ in range(n_sub):
        pltpu.sync_copy(rows_hbm.at[pl.ds(s * per_w + k_ * SCH, SCH)],
                        rows_v)
        for j in range(ch_per_sub):
          pltpu.sync_copy(rows_v.at[pl.ds(j * CH, CH)],
                          acc.at[idx_v.at[k_ * ch_per_sub + j]], add=True)
      plsc.subcore_barrier()

      pltpu.sync_copy(acc.at[pl.ds(s * stripe, stripe)],
                      out_hbm.at[pl.ds(q * nh + s * stripe, stripe)])
      plsc.subcore_barrier()

  return k(rows, idx_byq)


def _tc_msg(ea, xsp, Wp):
  E = xsp.shape[0]
  ED_ = ea.shape[1]
  H_ = Wp.shape[1]
  D = (Wp.shape[0] // (ED_ + 1))
  BE = 1024

  def body(ea_ref, xs_ref, w_ref, o_ref):
    xs_b = xs_ref[:, :D]
    ea_b = ea_ref[...]
    z = jnp.concatenate(
        [ea_b[:, d][:, None] * xs_b for d in range(ED_)] + [xs_b], axis=1)
    m = lax.dot_general(z, w_ref[...], (((1,), (0,)), ((), ())),
                        preferred_element_type=jnp.float32)
    o_ref[...] = jnp.concatenate(
        [m, jnp.ones((BE, 1), jnp.float32),
         jnp.zeros((BE, PW - H_ - 1), jnp.float32)], axis=1)

  return pl.pallas_call(
      body,
      grid=(E // BE,),
      in_specs=[pl.BlockSpec((BE, ED_), lambda i: (i, 0)),
                pl.BlockSpec((BE, PW), lambda i: (i, 0)),
                pl.BlockSpec(Wp.shape, lambda i: (0, 0))],
      out_specs=pl.BlockSpec((BE, PW), lambda i: (i, 0)),
      out_shape=jax.ShapeDtypeStruct((E, PW), jnp.float32),
  )(ea, xsp, Wp)


def _tc_update(sums, xp, root, bias, gamma, beta):
  Nn = xp.shape[0]
  D, H_ = root.shape
  BN = 2048
  inv = float((1.0 + EPS) ** -0.5)

  def body(s_ref, x_ref, root_ref, b_ref, g_ref, be_ref, o_ref):
    ssum = s_ref[:, :H_]
    cnt = s_ref[:, H_:H_ + 1]
    aggr = ssum / jnp.maximum(cnt, 1.0)
    v = aggr + lax.dot_general(x_ref[:, :D], root_ref[...],
                               (((1,), (0,)), ((), ())),
                               preferred_element_type=jnp.float32)
    v = v + b_ref[...]
    h = jnp.maximum(v * (g_ref[...] * inv) + be_ref[...], 0.0)
    o_ref[...] = jnp.concatenate(
        [h, jnp.ones((BN, 1), jnp.float32),
         jnp.zeros((BN, PW - H_ - 1), jnp.float32)], axis=1)

  return pl.pallas_call(
      body,
      grid=(Nn // BN,),
      in_specs=[pl.BlockSpec((BN, PW), lambda i: (i, 0)),
                pl.BlockSpec((BN, PW), lambda i: (i, 0)),
                pl.BlockSpec((D, H_), lambda i: (0, 0)),
                pl.BlockSpec((1, H_), lambda i: (0, 0)),
                pl.BlockSpec((1, H_), lambda i: (0, 0)),
                pl.BlockSpec((1, H_), lambda i: (0, 0))],
      out_specs=pl.BlockSpec((BN, PW), lambda i: (i, 0)),
      out_shape=jax.ShapeDtypeStruct((Nn, PW), jnp.float32),
  )(sums, xp, root, bias.reshape(1, H_), gamma.reshape(1, H_),
    beta.reshape(1, H_))


def _tc_final(psums, W1, b1, W2, b2):
  G_ = psums.shape[0]
  H_, Hh = W1.shape
  O_ = W2.shape[1]

  def body(s_ref, w1_ref, b1_ref, w2_ref, b2_ref, o_ref):
    pool = s_ref[:, :H_] / jnp.maximum(s_ref[:, H_:H_ + 1], 1.0)
    hid = lax.dot_general(pool, w1_ref[...], (((1,), (0,)), ((), ())),
                          preferred_element_type=jnp.float32)
    hid = jnp.maximum(hid + b1_ref[...], 0.0)
    out = lax.dot_general(hid, w2_ref[...], (((1,), (0,)), ((), ())),
                          preferred_element_type=jnp.float32)
    o_ref[...] = out + b2_ref[...]

  return pl.pallas_call(
      body,
      out_shape=jax.ShapeDtypeStruct((G_, O_), jnp.float32),
  )(psums, W1, b1.reshape(1, Hh), W2, b2.reshape(1, O_))


def _split_idx(idx, nseg, nq):
  nh = nseg // nq
  parts = []
  for q in range(nq):
    lo = q * nh
    loc = idx - lo
    parts.append(jnp.where((idx >= lo) & (idx < lo + nh), loc, nh))
  return jnp.stack(parts).reshape(nq, idx.shape[0] // CH, CH)


def kernel(x, edge_index, edge_attr, batch, edge_nn1_W, edge_nn1_b,
           conv1_root, conv1_bias, bn1_gamma, bn1_beta, edge_nn2_W,
           edge_nn2_b, conv2_root, conv2_bias, bn2_gamma, bn2_beta,
           mlp_W1, mlp_b1, mlp_W2, mlp_b2):
  Nn, IN_ = x.shape
  E_ = edge_index.shape[1]
  ED_ = edge_attr.shape[1]
  H_ = conv1_root.shape[1]
  G_ = 512

  src2d = edge_index[0].reshape(E_ // CH, CH)
  dst_byq = _split_idx(edge_index[1], Nn, 2 * NC)
  bat_byq = _split_idx(batch, G_, NC)

  W1p = jnp.concatenate([edge_nn1_W.reshape(ED_ * IN_, H_),
                         edge_nn1_b.reshape(IN_, H_)], axis=0)
  W2p = jnp.concatenate([edge_nn2_W.reshape(ED_ * H_, H_),
                         edge_nn2_b.reshape(H_, H_)], axis=0)

  xp = jnp.pad(x, ((0, 0), (0, PW - IN_)))

  xs1 = _sc_gather(xp, src2d)
  msg1 = _tc_msg(edge_attr, xs1, W1p)
  sums1 = _sc_scatter(msg1, dst_byq, Nn)
  h1 = _tc_update(sums1, xp, conv1_root, conv1_bias, bn1_gamma, bn1_beta)

  xs2 = _sc_gather(h1, src2d)
  msg2 = _tc_msg(edge_attr, xs2, W2p)
  sums2 = _sc_scatter(msg2, dst_byq, Nn)
  h2 = _tc_update(sums2, h1, conv2_root, conv2_bias, bn2_gamma, bn2_beta)

  psums = _sc_scatter(h2, bat_byq, G_)
  return _tc_final(psums, mlp_W1, mlp_b1, mlp_W2, mlp_b2)


# speedup vs baseline: 2.8321x; 1.0576x over previous
import functools

import jax
import jax.numpy as jnp
from jax import lax
from jax.experimental import pallas as pl
from jax.experimental.pallas import tpu as pltpu
from jax.experimental.pallas import tpu_sc as plsc

NC = 2
NS = 16
NW = NC * NS
CH = 128
PW = 128
SCH = 512
SSCH = 256
EPS = 1e-05


def _vsc_mesh():
  return plsc.VectorSubcoreMesh(core_axis_name="c", subcore_axis_name="s",
                                num_cores=NC, num_subcores=NS)


def _sc_gather(table, idx2d):
  n_idx_rows, ch = idx2d.shape
  R = n_idx_rows * ch
  per_w = R // NW
  n_ch = per_w // CH
  n_sub = per_w // SCH
  ch_per_sub = SCH // CH

  @functools.partial(
      pl.kernel,
      out_type=jax.ShapeDtypeStruct((R, PW), jnp.float32),
      mesh=_vsc_mesh(),
      scratch_types=[
          pltpu.VMEM((n_ch, CH), jnp.int32),
          pltpu.VMEM((SCH, PW), jnp.float32),
          pltpu.SemaphoreType.DMA,
      ])
  def k(table_hbm, idx_hbm, out_hbm, idx_v, rows_v, sem):
    c = lax.axis_index("c")
    s = lax.axis_index("s")
    wid = c * NS + s
    pltpu.sync_copy(idx_hbm.at[pl.ds(wid * n_ch, n_ch)], idx_v)
    for k_ in range(n_sub):
      cps = []
      for j in range(ch_per_sub):
        cps.append(pltpu.async_copy(
            table_hbm.at[idx_v.at[k_ * ch_per_sub + j]],
            rows_v.at[pl.ds(j * CH, CH)], sem))
      for cp in cps:
        cp.wait()
      pltpu.sync_copy(rows_v, out_hbm.at[pl.ds(wid * per_w + k_ * SCH, SCH)])

  return k(table, idx2d)


def _sc_scatter(rows, idx_byq, nseg):
  R, W = rows.shape
  NQ = idx_byq.shape[0]
  NPH = NQ // NC
  nh = nseg // NQ
  stripe = nh // NS
  per_w = R // NS
  n_ch = per_w // CH
  n_sub = per_w // SSCH
  ch_per_sub = SSCH // CH
  zrows = min(16, stripe)
  zreps = stripe // zrows

  @functools.partial(
      pl.kernel,
      out_type=jax.ShapeDtypeStruct((nseg, W), jnp.float32),
      mesh=_vsc_mesh(),
      scratch_types=[
          pltpu.VMEM((n_ch, CH), jnp.int32),
          pltpu.VMEM((2, SSCH, W), jnp.float32),
          pltpu.VMEM((16, W), jnp.float32),
          pltpu.VMEM_SHARED((nh + 8, W), jnp.float32),
          pltpu.SemaphoreType.DMA,
          pltpu.SemaphoreType.DMA,
          pltpu.SemaphoreType.DMA,
          pltpu.SemaphoreType.DMA,
      ])
  def k(rows_hbm, idx_hbm, out_hbm, idx_v, rows_v, zer_v, acc,
        sem_s0, sem_s1, sem_i0, sem_i1):
    c = lax.axis_index("c")
    s = lax.axis_index("s")
    stage_sems = [sem_s0, sem_s1]
    ind_sems = [sem_i0, sem_i1]

    def fill_body(i, _):
      for j in range(W // 16):
        zer_v[i, pl.ds(16 * j, 16)] = jnp.zeros((16,), jnp.float32)
      return 0
    lax.fori_loop(0, 16, fill_body, 0)

    for p in range(NPH):
      q = c * NPH + p
      for z in range(zreps):
        pltpu.sync_copy(zer_v.at[pl.ds(0, zrows)],
                        acc.at[pl.ds(s * stripe + z * zrows, zrows)])
      plsc.subcore_barrier()

      pltpu.sync_copy(idx_hbm.at[q, pl.ds(s * n_ch, n_ch)], idx_v)
      stage_cp = [None, None]
      ind_cp = [[], []]

      def _stage(k_):
        b = k_ % 2
        stage_cp[b] = pltpu.async_copy(
            rows_hbm.at[pl.ds(s * per_w + k_ * SSCH, SSCH)], rows_v.at[b],
            stage_sems[b])

      _stage(0)
      for k_ in range(n_sub):
        b = k_ % 2
        stage_cp[b].wait()
        if k_ + 1 < n_sub:
          for cp in ind_cp[1 - b]:
            cp.wait()
          ind_cp[1 - b] = []
          _stage(k_ + 1)
        for j in range(ch_per_sub):
          ind_cp[b].append(pltpu.async_copy(
              rows_v.at[b, pl.ds(j * CH, CH)],
              acc.at[idx_v.at[k_ * ch_per_sub + j]], ind_sems[b], add=True))
      for b in (0, 1):
        for cp in ind_cp[b]:
          cp.wait()
      plsc.subcore_barrier()

      pltpu.sync_copy(acc.at[pl.ds(s * stripe, stripe)],
                      out_hbm.at[pl.ds(q * nh + s * stripe, stripe)])
      plsc.subcore_barrier()

  return k(rows, idx_byq)


def _tc_msg(ea, xsp, Wp):
  E = xsp.shape[0]
  ED_ = ea.shape[1]
  H_ = Wp.shape[1]
  D = (Wp.shape[0] // (ED_ + 1))
  BE = 1024

  def body(ea_ref, xs_ref, w_ref, o_ref):
    xs_b = xs_ref[:, :D]
    ea_b = ea_ref[...]
    z = jnp.concatenate(
        [ea_b[:, d][:, None] * xs_b for d in range(ED_)] + [xs_b], axis=1)
    m = lax.dot_general(z, w_ref[...], (((1,), (0,)), ((), ())),
                        preferred_element_type=jnp.float32)
    o_ref[...] = jnp.concatenate(
        [m, jnp.ones((BE, 1), jnp.float32),
         jnp.zeros((BE, PW - H_ - 1), jnp.float32)], axis=1)

  return pl.pallas_call(
      body,
      grid=(E // BE,),
      in_specs=[pl.BlockSpec((BE, ED_), lambda i: (i, 0)),
                pl.BlockSpec((BE, PW), lambda i: (i, 0)),
                pl.BlockSpec(Wp.shape, lambda i: (0, 0))],
      out_specs=pl.BlockSpec((BE, PW), lambda i: (i, 0)),
      out_shape=jax.ShapeDtypeStruct((E, PW), jnp.float32),
  )(ea, xsp, Wp)


def _tc_update(sums, xp, root, bias, gamma, beta):
  Nn = xp.shape[0]
  SW = sums.shape[1]
  D, H_ = root.shape
  BN = 2048
  inv = float((1.0 + EPS) ** -0.5)

  def body(s_ref, x_ref, root_ref, b_ref, g_ref, be_ref, o_ref):
    ssum = s_ref[:, :H_]
    cnt = s_ref[:, H_:H_ + 1]
    aggr = ssum / jnp.maximum(cnt, 1.0)
    v = aggr + lax.dot_general(x_ref[:, :D], root_ref[...],
                               (((1,), (0,)), ((), ())),
                               preferred_element_type=jnp.float32)
    v = v + b_ref[...]
    h = jnp.maximum(v * (g_ref[...] * inv) + be_ref[...], 0.0)
    o_ref[...] = jnp.concatenate(
        [h, jnp.ones((BN, 1), jnp.float32),
         jnp.zeros((BN, PW - H_ - 1), jnp.float32)], axis=1)

  return pl.pallas_call(
      body,
      grid=(Nn // BN,),
      in_specs=[pl.BlockSpec((BN, SW), lambda i: (i, 0)),
                pl.BlockSpec((BN, PW), lambda i: (i, 0)),
                pl.BlockSpec((D, H_), lambda i: (0, 0)),
                pl.BlockSpec((1, H_), lambda i: (0, 0)),
                pl.BlockSpec((1, H_), lambda i: (0, 0)),
                pl.BlockSpec((1, H_), lambda i: (0, 0))],
      out_specs=pl.BlockSpec((BN, PW), lambda i: (i, 0)),
      out_shape=jax.ShapeDtypeStruct((Nn, PW), jnp.float32),
  )(sums, xp, root, bias.reshape(1, H_), gamma.reshape(1, H_),
    beta.reshape(1, H_))


def _tc_final(psums, W1, b1, W2, b2):
  G_ = psums.shape[0]
  H_, Hh = W1.shape
  O_ = W2.shape[1]

  def body(s_ref, w1_ref, b1_ref, w2_ref, b2_ref, o_ref):
    pool = s_ref[:, :H_] / jnp.maximum(s_ref[:, H_:H_ + 1], 1.0)
    hid = lax.dot_general(pool, w1_ref[...], (((1,), (0,)), ((), ())),
                          preferred_element_type=jnp.float32)
    hid = jnp.maximum(hid + b1_ref[...], 0.0)
    out = lax.dot_general(hid, w2_ref[...], (((1,), (0,)), ((), ())),
                          preferred_element_type=jnp.float32)
    o_ref[...] = out + b2_ref[...]

  return pl.pallas_call(
      body,
      out_shape=jax.ShapeDtypeStruct((G_, O_), jnp.float32),
  )(psums, W1, b1.reshape(1, Hh), W2, b2.reshape(1, O_))


def _split_idx(idx, nseg, nq):
  nh = nseg // nq
  parts = []
  for q in range(nq):
    lo = q * nh
    loc = idx - lo
    parts.append(jnp.where((idx >= lo) & (idx < lo + nh), loc, nh))
  return jnp.stack(parts).reshape(nq, idx.shape[0] // CH, CH)


def kernel(x, edge_index, edge_attr, batch, edge_nn1_W, edge_nn1_b,
           conv1_root, conv1_bias, bn1_gamma, bn1_beta, edge_nn2_W,
           edge_nn2_b, conv2_root, conv2_bias, bn2_gamma, bn2_beta,
           mlp_W1, mlp_b1, mlp_W2, mlp_b2):
  Nn, IN_ = x.shape
  E_ = edge_index.shape[1]
  ED_ = edge_attr.shape[1]
  H_ = conv1_root.shape[1]
  G_ = 512

  src2d = edge_index[0].reshape(E_ // CH, CH)
  dst_byh = _split_idx(edge_index[1], Nn, NC)
  dst_byq = _split_idx(edge_index[1], Nn, 2 * NC)
  bat_byq = _split_idx(batch, G_, NC)

  W1p = jnp.concatenate([edge_nn1_W.reshape(ED_ * IN_, H_),
                         edge_nn1_b.reshape(IN_, H_)], axis=0)
  W2p = jnp.concatenate([edge_nn2_W.reshape(ED_ * H_, H_),
                         edge_nn2_b.reshape(H_, H_)], axis=0)

  xp = jnp.pad(x, ((0, 0), (0, PW - IN_)))

  xs1 = _sc_gather(xp, src2d)
  msg1 = _tc_msg(edge_attr, xs1, W1p)
  sums1 = _sc_scatter(msg1, dst_byq, Nn)
  h1 = _tc_update(sums1, xp, conv1_root, conv1_bias, bn1_gamma, bn1_beta)

  xs2 = _sc_gather(h1, src2d)
  msg2 = _tc_msg(edge_attr, xs2, W2p)
  sums2 = _sc_scatter(msg2, dst_byq, Nn)
  h2 = _tc_update(sums2, h1, conv2_root, conv2_bias, bn2_gamma, bn2_beta)

  psums = _sc_scatter(h2, bat_byq, G_)
  return _tc_final(psums, mlp_W1, mlp_b1, mlp_W2, mlp_b2)


# speedup vs baseline: 3.1793x; 1.1226x over previous
import functools

import jax
import jax.numpy as jnp
from jax import lax
from jax.experimental import pallas as pl
from jax.experimental.pallas import tpu as pltpu
from jax.experimental.pallas import tpu_sc as plsc

NC = 2
NS = 16
NW = NC * NS
CH = 128
PW = 128
SCH = 512
SSCH = 256
EPS = 1e-05


def _vsc_mesh():
  return plsc.VectorSubcoreMesh(core_axis_name="c", subcore_axis_name="s",
                                num_cores=NC, num_subcores=NS)


def _sc_gather(table, idx2d):
  n_idx_rows, ch = idx2d.shape
  R = n_idx_rows * ch
  per_w = R // NW
  n_ch = per_w // CH
  n_sub = per_w // SCH
  ch_per_sub = SCH // CH

  @functools.partial(
      pl.kernel,
      out_type=jax.ShapeDtypeStruct((R, PW), jnp.float32),
      mesh=_vsc_mesh(),
      scratch_types=[
          pltpu.VMEM((n_ch, CH), jnp.int32),
          pltpu.VMEM((SCH, PW), jnp.float32),
          pltpu.SemaphoreType.DMA,
      ])
  def k(table_hbm, idx_hbm, out_hbm, idx_v, rows_v, sem):
    c = lax.axis_index("c")
    s = lax.axis_index("s")
    wid = c * NS + s
    pltpu.sync_copy(idx_hbm.at[pl.ds(wid * n_ch, n_ch)], idx_v)
    for k_ in range(n_sub):
      cps = []
      for j in range(ch_per_sub):
        cps.append(pltpu.async_copy(
            table_hbm.at[idx_v.at[k_ * ch_per_sub + j]],
            rows_v.at[pl.ds(j * CH, CH)], sem))
      for cp in cps:
        cp.wait()
      pltpu.sync_copy(rows_v, out_hbm.at[pl.ds(wid * per_w + k_ * SCH, SCH)])

  return k(table, idx2d)


def _sc_scatter(rows, idx_byq, nseg):
  R, W = rows.shape
  NQ = idx_byq.shape[0]
  NPH = NQ // NC
  nh = nseg // NQ
  stripe = nh // NS
  per_w = R // NS
  n_ch = per_w // CH
  n_sub = per_w // SSCH
  ch_per_sub = SSCH // CH
  zrows = min(16, stripe)
  zreps = stripe // zrows

  @functools.partial(
      pl.kernel,
      out_type=jax.ShapeDtypeStruct((nseg, W), jnp.float32),
      mesh=_vsc_mesh(),
      scratch_types=[
          pltpu.VMEM((n_ch, CH), jnp.int32),
          pltpu.VMEM((2, SSCH, W), jnp.float32),
          pltpu.VMEM((16, W), jnp.float32),
          pltpu.VMEM_SHARED((nh + NS, W), jnp.float32),
          pltpu.SemaphoreType.DMA,
          pltpu.SemaphoreType.DMA,
          pltpu.SemaphoreType.DMA,
          pltpu.SemaphoreType.DMA,
      ])
  def k(rows_hbm, idx_hbm, out_hbm, idx_v, rows_v, zer_v, acc,
        sem_s0, sem_s1, sem_i0, sem_i1):
    c = lax.axis_index("c")
    s = lax.axis_index("s")
    stage_sems = [sem_s0, sem_s1]
    ind_sems = [sem_i0, sem_i1]

    def fill_body(i, _):
      for j in range(W // 16):
        zer_v[i, pl.ds(16 * j, 16)] = jnp.zeros((16,), jnp.float32)
      return 0
    lax.fori_loop(0, 16, fill_body, 0)

    for p in range(NPH):
      q = c * NPH + p
      for z in range(zreps):
        pltpu.sync_copy(zer_v.at[pl.ds(0, zrows)],
                        acc.at[pl.ds(s * stripe + z * zrows, zrows)])
      plsc.subcore_barrier()

      pltpu.sync_copy(idx_hbm.at[q, pl.ds(s * n_ch, n_ch)], idx_v)
      stage_cp = [None, None]
      ind_cp = [[], []]

      def _stage(k_):
        b = k_ % 2
        stage_cp[b] = pltpu.async_copy(
            rows_hbm.at[pl.ds(s * per_w + k_ * SSCH, SSCH)], rows_v.at[b],
            stage_sems[b])

      _stage(0)
      for k_ in range(n_sub):
        b = k_ % 2
        stage_cp[b].wait()
        if k_ + 1 < n_sub:
          for cp in ind_cp[1 - b]:
            cp.wait()
          ind_cp[1 - b] = []
          _stage(k_ + 1)
        for j in range(ch_per_sub):
          ind_cp[b].append(pltpu.async_copy(
              rows_v.at[b, pl.ds(j * CH, CH)],
              acc.at[idx_v.at[k_ * ch_per_sub + j]], ind_sems[b], add=True))
      for b in (0, 1):
        for cp in ind_cp[b]:
          cp.wait()
      plsc.subcore_barrier()

      pltpu.sync_copy(acc.at[pl.ds(s * stripe, stripe)],
                      out_hbm.at[pl.ds(q * nh + s * stripe, stripe)])
      plsc.subcore_barrier()

  return k(rows, idx_byq)


def _tc_msg(ea, xsp, Wp):
  E = xsp.shape[0]
  ED_ = ea.shape[1]
  H_ = Wp.shape[1]
  D = (Wp.shape[0] // (ED_ + 1))
  BE = 1024

  def body(ea_ref, xs_ref, w_ref, o_ref):
    xs_b = xs_ref[:, :D]
    ea_b = ea_ref[...]
    z = jnp.concatenate(
        [ea_b[:, d][:, None] * xs_b for d in range(ED_)] + [xs_b], axis=1)
    m = lax.dot_general(z, w_ref[...], (((1,), (0,)), ((), ())),
                        preferred_element_type=jnp.float32)
    o_ref[...] = jnp.concatenate(
        [m, jnp.ones((BE, 1), jnp.float32),
         jnp.zeros((BE, PW - H_ - 1), jnp.float32)], axis=1)

  return pl.pallas_call(
      body,
      grid=(E // BE,),
      in_specs=[pl.BlockSpec((BE, ED_), lambda i: (i, 0)),
                pl.BlockSpec((BE, PW), lambda i: (i, 0)),
                pl.BlockSpec(Wp.shape, lambda i: (0, 0))],
      out_specs=pl.BlockSpec((BE, PW), lambda i: (i, 0)),
      out_shape=jax.ShapeDtypeStruct((E, PW), jnp.float32),
  )(ea, xsp, Wp)


def _tc_update(sums, xp, root, bias, gamma, beta):
  Nn = xp.shape[0]
  SW = sums.shape[1]
  D, H_ = root.shape
  BN = 2048
  inv = float((1.0 + EPS) ** -0.5)

  def body(s_ref, x_ref, root_ref, b_ref, g_ref, be_ref, o_ref):
    ssum = s_ref[:, :H_]
    cnt = s_ref[:, H_:H_ + 1]
    aggr = ssum / jnp.maximum(cnt, 1.0)
    v = aggr + lax.dot_general(x_ref[:, :D], root_ref[...],
                               (((1,), (0,)), ((), ())),
                               preferred_element_type=jnp.float32)
    v = v + b_ref[...]
    h = jnp.maximum(v * (g_ref[...] * inv) + be_ref[...], 0.0)
    o_ref[...] = jnp.concatenate(
        [h, jnp.ones((BN, 1), jnp.float32),
         jnp.zeros((BN, PW - H_ - 1), jnp.float32)], axis=1)

  return pl.pallas_call(
      body,
      grid=(Nn // BN,),
      in_specs=[pl.BlockSpec((BN, SW), lambda i: (i, 0)),
                pl.BlockSpec((BN, PW), lambda i: (i, 0)),
                pl.BlockSpec((D, H_), lambda i: (0, 0)),
                pl.BlockSpec((1, H_), lambda i: (0, 0)),
                pl.BlockSpec((1, H_), lambda i: (0, 0)),
                pl.BlockSpec((1, H_), lambda i: (0, 0))],
      out_specs=pl.BlockSpec((BN, PW), lambda i: (i, 0)),
      out_shape=jax.ShapeDtypeStruct((Nn, PW), jnp.float32),
  )(sums, xp, root, bias.reshape(1, H_), gamma.reshape(1, H_),
    beta.reshape(1, H_))


def _tc_final(psums, W1, b1, W2, b2):
  G_ = psums.shape[0]
  H_, Hh = W1.shape
  O_ = W2.shape[1]

  def body(s_ref, w1_ref, b1_ref, w2_ref, b2_ref, o_ref):
    pool = s_ref[:, :H_] / jnp.maximum(s_ref[:, H_:H_ + 1], 1.0)
    hid = lax.dot_general(pool, w1_ref[...], (((1,), (0,)), ((), ())),
                          preferred_element_type=jnp.float32)
    hid = jnp.maximum(hid + b1_ref[...], 0.0)
    out = lax.dot_general(hid, w2_ref[...], (((1,), (0,)), ((), ())),
                          preferred_element_type=jnp.float32)
    o_ref[...] = out + b2_ref[...]

  return pl.pallas_call(
      body,
      out_shape=jax.ShapeDtypeStruct((G_, O_), jnp.float32),
  )(psums, W1, b1.reshape(1, Hh), W2, b2.reshape(1, O_))


def _split_idx(idx, nseg, nq):
  nh = nseg // nq
  R = idx.shape[0]
  dump = nh + (jnp.arange(R, dtype=jnp.int32) // (R // NS))
  parts = []
  for q in range(nq):
    lo = q * nh
    loc = idx - lo
    parts.append(jnp.where((idx >= lo) & (idx < lo + nh), loc, dump))
  return jnp.stack(parts).reshape(nq, R // CH, CH)


def kernel(x, edge_index, edge_attr, batch, edge_nn1_W, edge_nn1_b,
           conv1_root, conv1_bias, bn1_gamma, bn1_beta, edge_nn2_W,
           edge_nn2_b, conv2_root, conv2_bias, bn2_gamma, bn2_beta,
           mlp_W1, mlp_b1, mlp_W2, mlp_b2):
  Nn, IN_ = x.shape
  E_ = edge_index.shape[1]
  ED_ = edge_attr.shape[1]
  H_ = conv1_root.shape[1]
  G_ = 512

  src2d = edge_index[0].reshape(E_ // CH, CH)
  dst_byh = _split_idx(edge_index[1], Nn, NC)
  dst_byq = _split_idx(edge_index[1], Nn, 2 * NC)
  bat_byq = _split_idx(batch, G_, NC)

  W1p = jnp.concatenate([edge_nn1_W.reshape(ED_ * IN_, H_),
                         edge_nn1_b.reshape(IN_, H_)], axis=0)
  W2p = jnp.concatenate([edge_nn2_W.reshape(ED_ * H_, H_),
                         edge_nn2_b.reshape(H_, H_)], axis=0)

  xp = jnp.pad(x, ((0, 0), (0, PW - IN_)))

  xs1 = _sc_gather(xp, src2d)
  msg1 = _tc_msg(edge_attr, xs1, W1p)
  sums1 = _sc_scatter(msg1, dst_byq, Nn)
  h1 = _tc_update(sums1, xp, conv1_root, conv1_bias, bn1_gamma, bn1_beta)

  xs2 = _sc_gather(h1, src2d)
  msg2 = _tc_msg(edge_attr, xs2, W2p)
  sums2 = _sc_scatter(msg2, dst_byq, Nn)
  h2 = _tc_update(sums2, h1, conv2_root, conv2_bias, bn2_gamma, bn2_beta)

  psums = _sc_scatter(h2, bat_byq, G_)
  return _tc_final(psums, mlp_W1, mlp_b1, mlp_W2, mlp_b2)


# speedup vs baseline: 3.9554x; 1.2441x over previous
import functools

import jax
import jax.numpy as jnp
from jax import lax
from jax.experimental import pallas as pl
from jax.experimental.pallas import tpu as pltpu
from jax.experimental.pallas import tpu_sc as plsc

NC = 2
NS = 16
NW = NC * NS
CH = 128
PW = 128
SCH = 512
SSCH = 256
EPS = 1e-05


def _vsc_mesh():
  return plsc.VectorSubcoreMesh(core_axis_name="c", subcore_axis_name="s",
                                num_cores=NC, num_subcores=NS)


def _sc_gather(table, idx2d):
  n_idx_rows, ch = idx2d.shape
  R = n_idx_rows * ch
  per_w = R // NW
  n_ch = per_w // CH
  n_sub = per_w // SCH
  ch_per_sub = SCH // CH

  @functools.partial(
      pl.kernel,
      out_type=jax.ShapeDtypeStruct((R, PW), jnp.float32),
      mesh=_vsc_mesh(),
      scratch_types=[
          pltpu.VMEM((n_ch, CH), jnp.int32),
          pltpu.VMEM((SCH, PW), jnp.float32),
          pltpu.SemaphoreType.DMA,
      ])
  def k(table_hbm, idx_hbm, out_hbm, idx_v, rows_v, sem):
    c = lax.axis_index("c")
    s = lax.axis_index("s")
    wid = c * NS + s
    pltpu.sync_copy(idx_hbm.at[pl.ds(wid * n_ch, n_ch)], idx_v)
    for k_ in range(n_sub):
      cps = []
      for j in range(ch_per_sub):
        cps.append(pltpu.async_copy(
            table_hbm.at[idx_v.at[k_ * ch_per_sub + j]],
            rows_v.at[pl.ds(j * CH, CH)], sem))
      for cp in cps:
        cp.wait()
      pltpu.sync_copy(rows_v, out_hbm.at[pl.ds(wid * per_w + k_ * SCH, SCH)])

  return k(table, idx2d)


def _sc_scatter(rows, idx_byq, nseg):
  R, W = rows.shape
  NQ = idx_byq.shape[0]
  NPH = NQ // NC
  nh = nseg // NQ
  stripe = nh // NS
  per_w = R // NS
  n_ch = per_w // CH
  n_sub = per_w // SSCH
  ch_per_sub = SSCH // CH
  zrows = min(16, stripe)
  zreps = stripe // zrows

  @functools.partial(
      pl.kernel,
      out_type=jax.ShapeDtypeStruct((nseg, W), jnp.float32),
      mesh=_vsc_mesh(),
      scratch_types=[
          pltpu.VMEM((n_ch, CH), jnp.int32),
          pltpu.VMEM((2, SSCH, W), jnp.float32),
          pltpu.VMEM((16, W), jnp.float32),
          pltpu.VMEM_SHARED((nh + NS, W), jnp.float32),
          pltpu.SemaphoreType.DMA,
          pltpu.SemaphoreType.DMA,
          pltpu.SemaphoreType.DMA,
          pltpu.SemaphoreType.DMA,
      ])
  def k(rows_hbm, idx_hbm, out_hbm, idx_v, rows_v, zer_v, acc,
        sem_s0, sem_s1, sem_i0, sem_i1):
    c = lax.axis_index("c")
    s = lax.axis_index("s")
    stage_sems = [sem_s0, sem_s1]
    ind_sems = [sem_i0, sem_i1]

    def fill_body(i, _):
      for j in range(W // 16):
        zer_v[i, pl.ds(16 * j, 16)] = jnp.zeros((16,), jnp.float32)
      return 0
    lax.fori_loop(0, 16, fill_body, 0)

    for p in range(NPH):
      q = c * NPH + p
      for z in range(zreps):
        pltpu.sync_copy(zer_v.at[pl.ds(0, zrows)],
                        acc.at[pl.ds(s * stripe + z * zrows, zrows)])
      plsc.subcore_barrier()

      pltpu.sync_copy(idx_hbm.at[q, pl.ds(s * n_ch, n_ch)], idx_v)
      stage_cp = [None, None]
      ind_cp = [[], []]

      def _stage(k_):
        b = k_ % 2
        stage_cp[b] = pltpu.async_copy(
            rows_hbm.at[pl.ds(s * per_w + k_ * SSCH, SSCH)], rows_v.at[b],
            stage_sems[b])

      _stage(0)
      for k_ in range(n_sub):
        b = k_ % 2
        stage_cp[b].wait()
        if k_ + 1 < n_sub:
          for cp in ind_cp[1 - b]:
            cp.wait()
          ind_cp[1 - b] = []
          _stage(k_ + 1)
        for j in range(ch_per_sub):
          ind_cp[b].append(pltpu.async_copy(
              rows_v.at[b, pl.ds(j * CH, CH)],
              acc.at[idx_v.at[k_ * ch_per_sub + j]], ind_sems[b], add=True))
      for b in (0, 1):
        for cp in ind_cp[b]:
          cp.wait()
      plsc.subcore_barrier()

      pltpu.sync_copy(acc.at[pl.ds(s * stripe, stripe)],
                      out_hbm.at[pl.ds(q * nh + s * stripe, stripe)])
      plsc.subcore_barrier()

  return k(rows, idx_byq)


def _tc_msg(ea1, xsp, Wh, S):
  E = xsp.shape[0]
  K = Wh.shape[1]
  H_ = K // (ea1.shape[1] // 2 + 1)
  ED_ = K // H_ - 1
  D = Wh.shape[0]
  BE = 2048

  def body(ea_ref, xs_ref, w_ref, s_ref, o_ref):
    y = lax.dot_general(xs_ref[:, :D], w_ref[...], (((1,), (0,)), ((), ())),
                        preferred_element_type=jnp.float32)
    e_exp = lax.dot_general(ea_ref[...], s_ref[...], (((1,), (0,)), ((), ())),
                            preferred_element_type=jnp.float32)
    m = y[:, ED_ * H_:]
    for d in range(ED_):
      m = m + y[:, d * H_:(d + 1) * H_] * e_exp[:, d * H_:(d + 1) * H_]
    o_ref[:, :H_] = m
    o_ref[:, H_:H_ + 1] = jnp.ones((BE, 1), jnp.float32)

  return pl.pallas_call(
      body,
      grid=(E // BE,),
      in_specs=[pl.BlockSpec((BE, ea1.shape[1]), lambda i: (i, 0)),
                pl.BlockSpec((BE, PW), lambda i: (i, 0)),
                pl.BlockSpec(Wh.shape, lambda i: (0, 0)),
                pl.BlockSpec(S.shape, lambda i: (0, 0))],
      out_specs=pl.BlockSpec((BE, PW), lambda i: (i, 0)),
      out_shape=jax.ShapeDtypeStruct((E, PW), jnp.float32),
  )(ea1, xsp, Wh, S)


def _tc_update(sums, xp, root, bias, gamma, beta):
  Nn = xp.shape[0]
  SW = sums.shape[1]
  D, H_ = root.shape
  BN = 2048
  inv = float((1.0 + EPS) ** -0.5)

  def body(s_ref, x_ref, root_ref, b_ref, g_ref, be_ref, o_ref):
    ssum = s_ref[:, :H_]
    cnt = s_ref[:, H_:H_ + 1]
    aggr = ssum / jnp.maximum(cnt, 1.0)
    v = aggr + lax.dot_general(x_ref[:, :D], root_ref[...],
                               (((1,), (0,)), ((), ())),
                               preferred_element_type=jnp.float32)
    v = v + b_ref[...]
    h = jnp.maximum(v * (g_ref[...] * inv) + be_ref[...], 0.0)
    o_ref[...] = jnp.concatenate(
        [h, jnp.ones((BN, 1), jnp.float32),
         jnp.zeros((BN, PW - H_ - 1), jnp.float32)], axis=1)

  return pl.pallas_call(
      body,
      grid=(Nn // BN,),
      in_specs=[pl.BlockSpec((BN, SW), lambda i: (i, 0)),
                pl.BlockSpec((BN, PW), lambda i: (i, 0)),
                pl.BlockSpec((D, H_), lambda i: (0, 0)),
                pl.BlockSpec((1, H_), lambda i: (0, 0)),
                pl.BlockSpec((1, H_), lambda i: (0, 0)),
                pl.BlockSpec((1, H_), lambda i: (0, 0))],
      out_specs=pl.BlockSpec((BN, PW), lambda i: (i, 0)),
      out_shape=jax.ShapeDtypeStruct((Nn, PW), jnp.float32),
  )(sums, xp, root, bias.reshape(1, H_), gamma.reshape(1, H_),
    beta.reshape(1, H_))


def _tc_final(psums, W1, b1, W2, b2):
  G_ = psums.shape[0]
  H_, Hh = W1.shape
  O_ = W2.shape[1]

  def body(s_ref, w1_ref, b1_ref, w2_ref, b2_ref, o_ref):
    pool = s_ref[:, :H_] / jnp.maximum(s_ref[:, H_:H_ + 1], 1.0)
    hid = lax.dot_general(pool, w1_ref[...], (((1,), (0,)), ((), ())),
                          preferred_element_type=jnp.float32)
    hid = jnp.maximum(hid + b1_ref[...], 0.0)
    out = lax.dot_general(hid, w2_ref[...], (((1,), (0,)), ((), ())),
                          preferred_element_type=jnp.float32)
    o_ref[...] = out + b2_ref[...]

  return pl.pallas_call(
      body,
      out_shape=jax.ShapeDtypeStruct((G_, O_), jnp.float32),
  )(psums, W1, b1.reshape(1, Hh), W2, b2.reshape(1, O_))


def _split_idx(idx, nseg, nq):
  nh = nseg // nq
  R = idx.shape[0]
  dump = nh + (jnp.arange(R, dtype=jnp.int32) // (R // NS))
  parts = []
  for q in range(nq):
    lo = q * nh
    loc = idx - lo
    parts.append(jnp.where((idx >= lo) & (idx < lo + nh), loc, dump))
  return jnp.stack(parts).reshape(nq, R // CH, CH)


def kernel(x, edge_index, edge_attr, batch, edge_nn1_W, edge_nn1_b,
           conv1_root, conv1_bias, bn1_gamma, bn1_beta, edge_nn2_W,
           edge_nn2_b, conv2_root, conv2_bias, bn2_gamma, bn2_beta,
           mlp_W1, mlp_b1, mlp_W2, mlp_b2):
  Nn, IN_ = x.shape
  E_ = edge_index.shape[1]
  ED_ = edge_attr.shape[1]
  H_ = conv1_root.shape[1]
  G_ = 512

  src2d = edge_index[0].reshape(E_ // CH, CH)
  dst_byh = _split_idx(edge_index[1], Nn, NC)
  dst_byq = _split_idx(edge_index[1], Nn, 2 * NC)
  bat_byq = _split_idx(batch, G_, NC)

  W1h = jnp.concatenate(
      [edge_nn1_W.reshape(ED_, IN_, H_).transpose(1, 0, 2).reshape(
          IN_, ED_ * H_), edge_nn1_b.reshape(IN_, H_)], axis=1)
  W2h = jnp.concatenate(
      [edge_nn2_W.reshape(ED_, H_, H_).transpose(1, 0, 2).reshape(
          H_, ED_ * H_), edge_nn2_b.reshape(H_, H_)], axis=1)
  ea1 = jnp.pad(edge_attr, ((0, 0), (0, 32 - ED_)))
  grp = jnp.arange((ED_ + 1) * H_, dtype=jnp.int32) // H_
  S = (grp[None, :] == jnp.arange(32, dtype=jnp.int32)[:, None]).astype(
      jnp.float32)

  xp = jnp.pad(x, ((0, 0), (0, PW - IN_)))

  xs1 = _sc_gather(xp, src2d)
  msg1 = _tc_msg(ea1, xs1, W1h, S)
  sums1 = _sc_scatter(msg1, dst_byq, Nn)
  h1 = _tc_update(sums1, xp, conv1_root, conv1_bias, bn1_gamma, bn1_beta)

  xs2 = _sc_gather(h1, src2d)
  msg2 = _tc_msg(ea1, xs2, W2h, S)
  sums2 = _sc_scatter(msg2, dst_byq, Nn)
  h2 = _tc_update(sums2, h1, conv2_root, conv2_bias, bn2_gamma, bn2_beta)

  psums = _sc_scatter(h2, bat_byq, G_)
  return _tc_final(psums, mlp_W1, mlp_b1, mlp_W2, mlp_b2)


# speedup vs baseline: 4.1431x; 1.0475x over previous
import functools

import jax
import jax.numpy as jnp
from jax import lax
from jax.experimental import pallas as pl
from jax.experimental.pallas import tpu as pltpu
from jax.experimental.pallas import tpu_sc as plsc

NC = 2
NS = 16
NW = NC * NS
CH = 128
PW = 128
SCH = 512
SSCH = 256
EPS = 1e-05


def _vsc_mesh():
  return plsc.VectorSubcoreMesh(core_axis_name="c", subcore_axis_name="s",
                                num_cores=NC, num_subcores=NS)


def _sc_gather(table, idx2d):
  n_idx_rows, ch = idx2d.shape
  R = n_idx_rows * ch
  per_w = R // NW
  n_ch = per_w // CH
  n_sub = per_w // SCH
  ch_per_sub = SCH // CH

  @functools.partial(
      pl.kernel,
      out_type=jax.ShapeDtypeStruct((R, PW), jnp.float32),
      mesh=_vsc_mesh(),
      scratch_types=[
          pltpu.VMEM((n_ch, CH), jnp.int32),
          pltpu.VMEM((SCH, PW), jnp.float32),
          pltpu.SemaphoreType.DMA,
      ])
  def k(table_hbm, idx_hbm, out_hbm, idx_v, rows_v, sem):
    c = lax.axis_index("c")
    s = lax.axis_index("s")
    wid = c * NS + s
    pltpu.sync_copy(idx_hbm.at[pl.ds(wid * n_ch, n_ch)], idx_v)
    for k_ in range(n_sub):
      cps = []
      for j in range(ch_per_sub):
        cps.append(pltpu.async_copy(
            table_hbm.at[idx_v.at[k_ * ch_per_sub + j]],
            rows_v.at[pl.ds(j * CH, CH)], sem))
      for cp in cps:
        cp.wait()
      pltpu.sync_copy(rows_v, out_hbm.at[pl.ds(wid * per_w + k_ * SCH, SCH)])

  return k(table, idx2d)


def _sc_scatter(rows, idx_byq, nseg):
  R, W = rows.shape
  NQ = idx_byq.shape[0]
  NPH = NQ // NC
  nh = nseg // NQ
  stripe = nh // NS
  per_w = R // NS
  n_ch = per_w // CH
  n_sub = per_w // SSCH
  ch_per_sub = SSCH // CH
  zrows = min(16, stripe)
  zreps = stripe // zrows

  @functools.partial(
      pl.kernel,
      out_type=jax.ShapeDtypeStruct((nseg, W), jnp.float32),
      mesh=_vsc_mesh(),
      scratch_types=[
          pltpu.VMEM((n_ch, CH), jnp.int32),
          pltpu.VMEM((2, SSCH, W), jnp.float32),
          pltpu.VMEM((16, W), jnp.float32),
          pltpu.VMEM_SHARED((nh + NS, W), jnp.float32),
          pltpu.SemaphoreType.DMA,
          pltpu.SemaphoreType.DMA,
          pltpu.SemaphoreType.DMA,
          pltpu.SemaphoreType.DMA,
      ])
  def k(rows_hbm, idx_hbm, out_hbm, idx_v, rows_v, zer_v, acc,
        sem_s0, sem_s1, sem_i0, sem_i1):
    c = lax.axis_index("c")
    s = lax.axis_index("s")
    stage_sems = [sem_s0, sem_s1]
    ind_sems = [sem_i0, sem_i1]

    def fill_body(i, _):
      for j in range(W // 16):
        zer_v[i, pl.ds(16 * j, 16)] = jnp.zeros((16,), jnp.float32)
      return 0
    lax.fori_loop(0, 16, fill_body, 0)

    for p in range(NPH):
      q = c * NPH + p
      for z in range(zreps):
        pltpu.sync_copy(zer_v.at[pl.ds(0, zrows)],
                        acc.at[pl.ds(s * stripe + z * zrows, zrows)])
      plsc.subcore_barrier()

      pltpu.sync_copy(idx_hbm.at[q, pl.ds(s * n_ch, n_ch)], idx_v)
      stage_cp = [None, None]
      ind_cp = [[], []]

      def _stage(k_):
        b = k_ % 2
        stage_cp[b] = pltpu.async_copy(
            rows_hbm.at[pl.ds(s * per_w + k_ * SSCH, SSCH)], rows_v.at[b],
            stage_sems[b])

      _stage(0)
      for k_ in range(n_sub):
        b = k_ % 2
        stage_cp[b].wait()
        if k_ + 1 < n_sub:
          for cp in ind_cp[1 - b]:
            cp.wait()
          ind_cp[1 - b] = []
          _stage(k_ + 1)
        for j in range(ch_per_sub):
          ind_cp[b].append(pltpu.async_copy(
              rows_v.at[b, pl.ds(j * CH, CH)],
              acc.at[idx_v.at[k_ * ch_per_sub + j]], ind_sems[b], add=True))
      for b in (0, 1):
        for cp in ind_cp[b]:
          cp.wait()
      plsc.subcore_barrier()

      pltpu.sync_copy(acc.at[pl.ds(s * stripe, stripe)],
                      out_hbm.at[pl.ds(q * nh + s * stripe, stripe)])
      plsc.subcore_barrier()

  return k(rows, idx_byq)


def _tc_msg(ea1, xsp, Wh, S, hf):
  E = xsp.shape[0] // 2
  K = Wh.shape[1]
  H_ = K // (ea1.shape[1] // 2 + 1)
  ED_ = K // H_ - 1
  D = Wh.shape[0]
  BE = 2048

  def body(ea_ref, xs_ref, w_ref, s_ref, o_ref):
    y = lax.dot_general(xs_ref[:, :D], w_ref[...], (((1,), (0,)), ((), ())),
                        preferred_element_type=jnp.float32)
    e_exp = lax.dot_general(ea_ref[...], s_ref[...], (((1,), (0,)), ((), ())),
                            preferred_element_type=jnp.float32)
    m = y[:, ED_ * H_:]
    for d in range(ED_):
      m = m + y[:, d * H_:(d + 1) * H_] * e_exp[:, d * H_:(d + 1) * H_]
    o_ref[:, :H_] = m
    o_ref[:, H_:H_ + 1] = jnp.ones((BE, 1), jnp.float32)

  nblk = E // BE
  off = hf * nblk

  return pl.pallas_call(
      body,
      grid=(nblk,),
      in_specs=[pl.BlockSpec((BE, ea1.shape[1]), lambda i: (i + off, 0)),
                pl.BlockSpec((BE, PW), lambda i: (i + off, 0)),
                pl.BlockSpec(Wh.shape, lambda i: (0, 0)),
                pl.BlockSpec(S.shape, lambda i: (0, 0))],
      out_specs=pl.BlockSpec((BE, PW), lambda i: (i, 0)),
      out_shape=jax.ShapeDtypeStruct((E, PW), jnp.float32),
  )(ea1, xsp, Wh, S)


def _tc_update(sums_a, sums_b, xp, root, bias, gamma, beta):
  Nn = xp.shape[0]
  SW = sums_a.shape[1]
  D, H_ = root.shape
  BN = 2048
  inv = float((1.0 + EPS) ** -0.5)

  def body(sa_ref, sb_ref, x_ref, root_ref, b_ref, g_ref, be_ref, o_ref):
    stot = sa_ref[...] + sb_ref[...]
    ssum = stot[:, :H_]
    cnt = stot[:, H_:H_ + 1]
    aggr = ssum / jnp.maximum(cnt, 1.0)
    v = aggr + lax.dot_general(x_ref[:, :D], root_ref[...],
                               (((1,), (0,)), ((), ())),
                               preferred_element_type=jnp.float32)
    v = v + b_ref[...]
    h = jnp.maximum(v * (g_ref[...] * inv) + be_ref[...], 0.0)
    o_ref[...] = jnp.concatenate(
        [h, jnp.ones((BN, 1), jnp.float32),
         jnp.zeros((BN, PW - H_ - 1), jnp.float32)], axis=1)

  return pl.pallas_call(
      body,
      grid=(Nn // BN,),
      in_specs=[pl.BlockSpec((BN, SW), lambda i: (i, 0)),
                pl.BlockSpec((BN, SW), lambda i: (i, 0)),
                pl.BlockSpec((BN, PW), lambda i: (i, 0)),
                pl.BlockSpec((D, H_), lambda i: (0, 0)),
                pl.BlockSpec((1, H_), lambda i: (0, 0)),
                pl.BlockSpec((1, H_), lambda i: (0, 0)),
                pl.BlockSpec((1, H_), lambda i: (0, 0))],
      out_specs=pl.BlockSpec((BN, PW), lambda i: (i, 0)),
      out_shape=jax.ShapeDtypeStruct((Nn, PW), jnp.float32),
  )(sums_a, sums_b, xp, root, bias.reshape(1, H_), gamma.reshape(1, H_),
    beta.reshape(1, H_))


def _tc_final(psums, W1, b1, W2, b2):
  G_ = psums.shape[0]
  H_, Hh = W1.shape
  O_ = W2.shape[1]

  def body(s_ref, w1_ref, b1_ref, w2_ref, b2_ref, o_ref):
    pool = s_ref[:, :H_] / jnp.maximum(s_ref[:, H_:H_ + 1], 1.0)
    hid = lax.dot_general(pool, w1_ref[...], (((1,), (0,)), ((), ())),
                          preferred_element_type=jnp.float32)
    hid = jnp.maximum(hid + b1_ref[...], 0.0)
    out = lax.dot_general(hid, w2_ref[...], (((1,), (0,)), ((), ())),
                          preferred_element_type=jnp.float32)
    o_ref[...] = out + b2_ref[...]

  return pl.pallas_call(
      body,
      out_shape=jax.ShapeDtypeStruct((G_, O_), jnp.float32),
  )(psums, W1, b1.reshape(1, Hh), W2, b2.reshape(1, O_))


def _split_idx(idx, nseg, nq):
  nh = nseg // nq
  R = idx.shape[0]
  dump = nh + (jnp.arange(R, dtype=jnp.int32) // (R // NS))
  parts = []
  for q in range(nq):
    lo = q * nh
    loc = idx - lo
    parts.append(jnp.where((idx >= lo) & (idx < lo + nh), loc, dump))
  return jnp.stack(parts).reshape(nq, R // CH, CH)


def kernel(x, edge_index, edge_attr, batch, edge_nn1_W, edge_nn1_b,
           conv1_root, conv1_bias, bn1_gamma, bn1_beta, edge_nn2_W,
           edge_nn2_b, conv2_root, conv2_bias, bn2_gamma, bn2_beta,
           mlp_W1, mlp_b1, mlp_W2, mlp_b2):
  Nn, IN_ = x.shape
  E_ = edge_index.shape[1]
  ED_ = edge_attr.shape[1]
  H_ = conv1_root.shape[1]
  G_ = 512

  src2d = edge_index[0].reshape(E_ // CH, CH)
  Eh = E_ // 2
  dsta_byq = _split_idx(edge_index[1][:Eh], Nn, 2 * NC)
  dstb_byq = _split_idx(edge_index[1][Eh:], Nn, 2 * NC)
  bat_byq = _split_idx(batch, G_, NC)

  W1h = jnp.concatenate(
      [edge_nn1_W.reshape(ED_, IN_, H_).transpose(1, 0, 2).reshape(
          IN_, ED_ * H_), edge_nn1_b.reshape(IN_, H_)], axis=1)
  W2h = jnp.concatenate(
      [edge_nn2_W.reshape(ED_, H_, H_).transpose(1, 0, 2).reshape(
          H_, ED_ * H_), edge_nn2_b.reshape(H_, H_)], axis=1)
  ea1 = jnp.pad(edge_attr, ((0, 0), (0, 32 - ED_)))
  grp = jnp.arange((ED_ + 1) * H_, dtype=jnp.int32) // H_
  S = (grp[None, :] == jnp.arange(32, dtype=jnp.int32)[:, None]).astype(
      jnp.float32)

  xp = jnp.pad(x, ((0, 0), (0, PW - IN_)))

  xs1 = _sc_gather(xp, src2d)
  msg1a = _tc_msg(ea1, xs1, W1h, S, 0)
  sums1a = _sc_scatter(msg1a, dsta_byq, Nn)
  msg1b = _tc_msg(ea1, xs1, W1h, S, 1)
  sums1b = _sc_scatter(msg1b, dstb_byq, Nn)
  h1 = _tc_update(sums1a, sums1b, xp, conv1_root, conv1_bias, bn1_gamma,
                  bn1_beta)

  xs2 = _sc_gather(h1, src2d)
  msg2a = _tc_msg(ea1, xs2, W2h, S, 0)
  sums2a = _sc_scatter(msg2a, dsta_byq, Nn)
  msg2b = _tc_msg(ea1, xs2, W2h, S, 1)
  sums2b = _sc_scatter(msg2b, dstb_byq, Nn)
  h2 = _tc_update(sums2a, sums2b, h1, conv2_root, conv2_bias, bn2_gamma,
                  bn2_beta)

  psums = _sc_scatter(h2, bat_byq, G_)
  return _tc_final(psums, mlp_W1, mlp_b1, mlp_W2, mlp_b2)
